# baseline (speedup 1.0000x reference)
import functools

import jax
import jax.numpy as jnp
from jax import lax
from jax.experimental import pallas as pl
from jax.experimental.pallas import tpu as pltpu
from jax.experimental.pallas import tpu_sc as plsc

N = 10000
NP = 10240
E = 160000
F_IN = 256
H = 256
C_OUT = 40
DH = 128
NSC = 2
NT = 16
CH = 128
ET = 10240
EP = NT * ET
NCH = ET // CH
RPT = NP // NT
T_TOTAL = 1.0
STEPS = 8
DT = T_TOTAL / STEPS

_ROW_BLK = 1024
_GRID_R = NP // _ROW_BLK


NCHT = EP // CH


def _lane_select(vec, s):
    sc = jnp.int32(0)
    for k in range(16):
        sc = sc + jnp.where(s == k, vec[k], 0)
    return sc


def _spmm_body(y2, eint, ewt, st_hbm, en_hbm, out2,
               rowbuf, g0, g1, e0, e1, w0, w1, stbuf, enbuf,
               sg0, sg1, si0, si1):
    c = lax.axis_index("c")
    s = lax.axis_index("s")
    base_row = s * RPT

    pltpu.sync_copy(st_hbm, stbuf)
    pltpu.sync_copy(en_hbm, enbuf)
    start = _lane_select(stbuf[...], s)
    end = _lane_select(enbuf[...], s)
    c0 = start // CH
    nch = (end + CH - 1) // CH - c0
    ebase = c * NCHT + c0

    zeros16 = jnp.zeros((16,), jnp.float32)

    def _zrow(i, carry):
        for f in range(8):
            rowbuf[i, pl.ds(16 * f, 16)] = zeros16
        return carry

    lax.fori_loop(0, RPT, _zrow, 0)

    gbufs = (g0, g1)
    ebufs = (e0, e1)
    wbufs = (w0, w1)
    gsems = (sg0, sg1)
    isems = (si0, si1)

    nclamp = jnp.maximum(nch - 1, 0)

    def _jc(j):
        return jnp.clip(j, 0, nclamp)

    for b in range(2):
        pltpu.async_copy(eint.at[ebase + _jc(b)], ebufs[b], isems[b])
        pltpu.async_copy(ewt.at[c0 + _jc(b)], wbufs[b], isems[b])
    pltpu.make_async_copy(eint.at[0], e0, si0).wait()
    pltpu.make_async_copy(ewt.at[0], w0, si0).wait()
    pltpu.async_copy(y2.at[e0.at[0]], g0, sg0)

    def _compute(gb, eb, wb):
        def _grp(g8, carry):
            rows16 = eb[1, pl.ds(16 * g8, 16)]
            w16 = wb[pl.ds(16 * g8, 16)]
            loc = rows16 - lax.broadcast(base_row, (16,))
            inr = (loc >= 0) & (loc < RPT)
            wm = jnp.where(inr, w16, 0.0)
            locc = jnp.clip(loc, 0, RPT - 1)
            for k in range(16):
                r = locc[k]
                wv = wm[k]
                e = g8 * 16 + k
                for f in range(8):
                    rowbuf[r, pl.ds(16 * f, 16)] = (
                        rowbuf[r, pl.ds(16 * f, 16)]
                        + wv * gb[e, pl.ds(16 * f, 16)])
            return carry

        lax.fori_loop(0, CH // 16, _grp, 0)

    def _pair(i2, carry):
        for b in range(2):
            j = i2 * 2 + b
            gb, eb, wb = gbufs[b], ebufs[b], wbufs[b]
            ob = 1 - b

            pltpu.make_async_copy(eint.at[0], ebufs[ob], isems[ob]).wait()
            pltpu.make_async_copy(ewt.at[0], wbufs[ob], isems[ob]).wait()
            pltpu.async_copy(y2.at[ebufs[ob].at[0]], gbufs[ob], gsems[ob])

            pltpu.make_async_copy(y2.at[pl.ds(0, CH)], gb, gsems[b]).wait()

            @pl.when(j < nch)
            def _do_compute():
                _compute(gb, eb, wb)

            pltpu.async_copy(eint.at[ebase + _jc(j + 2)], eb, isems[b])
            pltpu.async_copy(ewt.at[c0 + _jc(j + 2)], wb, isems[b])

        return carry

    lax.fori_loop(0, (nch + 1) // 2, _pair, 0)

    pltpu.make_async_copy(eint.at[0], e1, si1).wait()
    pltpu.make_async_copy(ewt.at[0], w1, si1).wait()
    pltpu.make_async_copy(y2.at[pl.ds(0, CH)], g0, sg0).wait()

    for j in range(RPT // CH):
        r0 = j * CH
        pltpu.sync_copy(rowbuf.at[pl.ds(r0, CH)],
                        out2.at[pl.ds(c * NP + base_row + r0, CH)])


_spmm_cached = None


def _spmm(y2, eint, ewt, starts, ends):
    global _spmm_cached
    if _spmm_cached is None:
        _spmm_cached = pl.kernel(
            _spmm_body,
            out_type=jax.ShapeDtypeStruct((NSC * NP, DH), jnp.float32),
            mesh=plsc.VectorSubcoreMesh(core_axis_name="c", subcore_axis_name="s",
                                        num_cores=NSC, num_subcores=NT),
            scratch_types=[
                pltpu.VMEM((RPT, DH), jnp.float32),
                pltpu.VMEM((CH, DH), jnp.float32),
                pltpu.VMEM((CH, DH), jnp.float32),
                pltpu.VMEM((2, CH), jnp.int32),
                pltpu.VMEM((2, CH), jnp.int32),
                pltpu.VMEM((CH,), jnp.float32),
                pltpu.VMEM((CH,), jnp.float32),
                pltpu.VMEM((16,), jnp.int32),
                pltpu.VMEM((16,), jnp.int32),
                pltpu.SemaphoreType.DMA,
                pltpu.SemaphoreType.DMA,
                pltpu.SemaphoreType.DMA,
                pltpu.SemaphoreType.DMA,
            ],
        )
    return _spmm_cached(y2, eint, ewt, starts, ends)


def _m1_body(x_ref, w1_ref, b1_ref, out_ref):
    out_ref[...] = (
        jnp.dot(x_ref[...], w1_ref[...], preferred_element_type=jnp.float32)
        + b1_ref[...]
    )


def _m1(xp, W1, b1):
    return pl.pallas_call(
        _m1_body,
        grid=(NSC, _GRID_R),
        in_specs=[
            pl.BlockSpec((_ROW_BLK, F_IN), lambda h, r: (r, 0)),
            pl.BlockSpec((F_IN, DH), lambda h, r: (0, h)),
            pl.BlockSpec((1, DH), lambda h, r: (0, h)),
        ],
        out_specs=pl.BlockSpec((_ROW_BLK, DH), lambda h, r: (h * _GRID_R + r, 0)),
        out_shape=jax.ShapeDtypeStruct((NSC * NP, DH), jnp.float32),
    )(xp, W1, b1.reshape(1, F_IN))


def _ew_stage_body(cs0, ck, cu, ax_ref, u_ref, y_ref, s_ref, h_ref, al_ref,
                   s_out, u_out):
    sig = 0.5 * jax.nn.sigmoid(al_ref[...])
    k = sig * (ax_ref[...] - u_ref[...]) + h_ref[...]
    s_out[...] = cs0 * s_ref[...] + ck * k
    u_out[...] = y_ref[...] + cu * k


def _ew_stage(cs0, ck, cu, ax, u, y, s, h, alpha):
    blk = pl.BlockSpec((_ROW_BLK, DH), lambda b: (b, 0))
    al_blk = pl.BlockSpec((_ROW_BLK, 1), lambda b: (b % _GRID_R, 0))
    return pl.pallas_call(
        functools.partial(_ew_stage_body, cs0, ck, cu),
        grid=(NSC * _GRID_R,),
        in_specs=[blk, blk, blk, blk, blk, al_blk],
        out_specs=[blk, blk],
        out_shape=[
            jax.ShapeDtypeStruct((NSC * NP, DH), jnp.float32),
            jax.ShapeDtypeStruct((NSC * NP, DH), jnp.float32),
        ],
    )(ax, u, y, s, h, alpha)


def _ew_final_body(ax_ref, u_ref, y_ref, s_ref, h_ref, al_ref, y_out):
    sig = 0.5 * jax.nn.sigmoid(al_ref[...])
    k = sig * (ax_ref[...] - u_ref[...]) + h_ref[...]
    y_out[...] = y_ref[...] + (DT / 6.0) * (s_ref[...] + k)


def _ew_final(ax, u, y, s, h, alpha):
    blk = pl.BlockSpec((_ROW_BLK, DH), lambda b: (b, 0))
    al_blk = pl.BlockSpec((_ROW_BLK, 1), lambda b: (b % _GRID_R, 0))
    return pl.pallas_call(
        _ew_final_body,
        grid=(NSC * _GRID_R,),
        in_specs=[blk, blk, blk, blk, blk, al_blk],
        out_specs=blk,
        out_shape=jax.ShapeDtypeStruct((NSC * NP, DH), jnp.float32),
    )(ax, u, y, s, h, alpha)


def _m2_body(ya_ref, yb_ref, w2_ref, b2_ref, out_ref):
    z = jax.nn.relu(jnp.concatenate([ya_ref[...], yb_ref[...]], axis=1))
    out_ref[...] = (
        jnp.dot(z, w2_ref[...], preferred_element_type=jnp.float32) + b2_ref[...]
    )


def _m2(y2, W2, b2):
    return pl.pallas_call(
        _m2_body,
        grid=(_GRID_R,),
        in_specs=[
            pl.BlockSpec((_ROW_BLK, DH), lambda r: (r, 0)),
            pl.BlockSpec((_ROW_BLK, DH), lambda r: (r + _GRID_R, 0)),
            pl.BlockSpec((H, C_OUT), lambda r: (0, 0)),
            pl.BlockSpec((1, C_OUT), lambda r: (0, 0)),
        ],
        out_specs=pl.BlockSpec((_ROW_BLK, C_OUT), lambda r: (r, 0)),
        out_shape=jax.ShapeDtypeStruct((N, C_OUT), jnp.float32),
    )(y2, y2, W2, b2.reshape(1, C_OUT))


def kernel(x, edge_index, edge_weight, W1, b1, alpha_train, W2, b2):
    row = edge_index[0]
    col = edge_index[1]
    pad = EP - E
    rows_s, cols_s, ws = lax.optimization_barrier(
        lax.sort((row, col, edge_weight), num_keys=1))
    rows_p = jnp.concatenate([rows_s, jnp.full((pad,), N, jnp.int32)])
    cols_p = jnp.concatenate([cols_s, jnp.zeros((pad,), jnp.int32)])
    ws_p = jnp.concatenate([ws, jnp.zeros((pad,), jnp.float32)])
    bounds = jnp.searchsorted(rows_p, jnp.arange(NT + 1, dtype=jnp.int32) * RPT)
    starts = bounds[:NT].astype(jnp.int32)
    ends = bounds[1:].astype(jnp.int32)
    rck = rows_p.reshape(NCHT, CH)
    eint = jnp.concatenate([
        jnp.stack([cols_p.reshape(NCHT, CH), rck], axis=1),
        jnp.stack([(cols_p + NP).reshape(NCHT, CH), rck], axis=1),
    ], axis=0)
    ewt = ws_p.reshape(NCHT, CH)
    xp = jnp.zeros((NP, F_IN), jnp.float32).at[:N].set(x)
    alpha = jnp.zeros((NP, 1), jnp.float32).at[:N, 0].set(alpha_train)

    h2 = _m1(xp, W1, b1)

    def _step(_, y):
        ax = _spmm(y, eint, ewt, starts, ends)
        s, u = _ew_stage(0.0, 1.0, DT / 2.0, ax, y, y, y, h2, alpha)
        ax = _spmm(u, eint, ewt, starts, ends)
        s, u = _ew_stage(1.0, 2.0, DT / 2.0, ax, u, y, s, h2, alpha)
        ax = _spmm(u, eint, ewt, starts, ends)
        s, u = _ew_stage(1.0, 2.0, DT, ax, u, y, s, h2, alpha)
        ax = _spmm(u, eint, ewt, starts, ends)
        return _ew_final(ax, u, y, s, h2, alpha)

    y = lax.fori_loop(0, STEPS, _step, h2)
    return _m2(y, W2, b2)


# speedup vs baseline: 1.3966x; 1.3966x over previous
import functools

import jax
import jax.numpy as jnp
from jax import lax
from jax.experimental import pallas as pl
from jax.experimental.pallas import tpu as pltpu
from jax.experimental.pallas import tpu_sc as plsc

N = 10000
NP = 10240
E = 160000
F_IN = 256
H = 256
C_OUT = 40
DH = 128
NSC = 2
NT = 16
CH = 128
ET = 10240
EP = NT * ET
NCH = ET // CH
RPT = NP // NT
T_TOTAL = 1.0
STEPS = 8
DT = T_TOTAL / STEPS

_ROW_BLK = 1024
_GRID_R = NP // _ROW_BLK


def _scale_chunk(gb, wall, j):
    def _grp(g, carry):
        w16 = wall[pl.ds(j * CH + 16 * g, 16)]
        for k in range(16):
            e = g * 16 + k
            wv = w16[k]
            for f in range(8):
                gb[e, pl.ds(16 * f, 16)] = gb[e, pl.ds(16 * f, 16)] * wv
        return carry

    lax.fori_loop(0, CH // 16, _grp, 0)


ET2 = ET // 2
NCH2 = NCH // 2


def _spmm_body(y2, colp2, rowp4, wp, out2, acc, g0, g1, call, rall, wall,
               sg0, sg1, ss0, ss1):
    c = lax.axis_index("c")
    s = lax.axis_index("s")
    coff = c * NP

    zeros16 = jnp.zeros((16,), jnp.float32)

    def _zrow(i, carry):
        for f in range(8):
            g0[i, pl.ds(16 * f, 16)] = zeros16
        return carry

    lax.fori_loop(0, CH, _zrow, 0)
    for j in range(RPT // CH):
        pltpu.sync_copy(g0, acc.at[pl.ds(s * RPT + j * CH, CH)])
    plsc.subcore_barrier()

    gbufs = (g0, g1)
    gsems = (sg0, sg1)
    ssems = (ss0, ss1)

    def _drain_g(buf, sem):
        pltpu.make_async_copy(y2.at[pl.ds(0, CH)], buf, sem).wait()

    for p in range(2):
        pltpu.sync_copy(colp2.at[pl.ds(c * EP + s * ET + p * ET2, ET2)], call)
        pltpu.sync_copy(rowp4.at[s * 2 + p], rall)
        pltpu.sync_copy(wp.at[pl.ds(s * ET + p * ET2, ET2)], wall)

        def _zg1(i, carry):
            for f in range(8):
                g1[i, pl.ds(16 * f, 16)] = zeros16
            return carry

        lax.fori_loop(0, CH, _zg1, 0)
        pltpu.async_copy(g1, acc.at[rall.at[0]], ss1, add=True)
        pltpu.async_copy(y2.at[call.at[pl.ds(0, CH)]], g0, sg0)

        def _pair(i2, carry):
            for b in range(2):
                j = i2 * 2 + b
                ob = 1 - b
                gb, sg, ssem = gbufs[b], gsems[b], ssems[b]
                _drain_g(gb, sg)
                _scale_chunk(gb, wall, j)
                pltpu.async_copy(gb, acc.at[rall.at[j]], ssem, add=True)
                _drain_g(gbufs[ob], ssems[ob])
                jn = jnp.minimum(j + 1, NCH2 - 1)
                pltpu.async_copy(y2.at[call.at[pl.ds(jn * CH, CH)]],
                                 gbufs[ob], gsems[ob])
            return carry

        lax.fori_loop(0, NCH2 // 2, _pair, 0)
        _drain_g(g1, ss1)
        _drain_g(g0, sg0)

    plsc.subcore_barrier()

    for j in range(RPT // CH):
        r0 = s * RPT + j * CH
        pltpu.sync_copy(acc.at[pl.ds(r0, CH)], g0)
        pltpu.sync_copy(g0, out2.at[pl.ds(coff + r0, CH)])


_spmm_cached = None


def _spmm(y2, colp2, rowp4, wp):
    global _spmm_cached
    if _spmm_cached is None:
        _spmm_cached = pl.kernel(
            _spmm_body,
            out_type=jax.ShapeDtypeStruct((NSC * NP, DH), jnp.float32),
            mesh=plsc.VectorSubcoreMesh(core_axis_name="c", subcore_axis_name="s",
                                        num_cores=NSC, num_subcores=NT),
            scratch_types=[
                pltpu.VMEM_SHARED((NP, DH), jnp.float32),
                pltpu.VMEM((CH, DH), jnp.float32),
                pltpu.VMEM((CH, DH), jnp.float32),
                pltpu.VMEM((ET2,), jnp.int32),
                pltpu.VMEM((NCH2, CH), jnp.int32),
                pltpu.VMEM((ET2,), jnp.float32),
                pltpu.SemaphoreType.DMA,
                pltpu.SemaphoreType.DMA,
                pltpu.SemaphoreType.DMA,
                pltpu.SemaphoreType.DMA,
            ],
        )
    return _spmm_cached(y2, colp2, rowp4, wp)


def _m1_body(x_ref, w1_ref, b1_ref, out_ref):
    out_ref[...] = (
        jnp.dot(x_ref[...], w1_ref[...], preferred_element_type=jnp.float32)
        + b1_ref[...]
    )


def _m1(xp, W1, b1):
    return pl.pallas_call(
        _m1_body,
        grid=(NSC, _GRID_R),
        in_specs=[
            pl.BlockSpec((_ROW_BLK, F_IN), lambda h, r: (r, 0)),
            pl.BlockSpec((F_IN, DH), lambda h, r: (0, h)),
            pl.BlockSpec((1, DH), lambda h, r: (0, h)),
        ],
        out_specs=pl.BlockSpec((_ROW_BLK, DH), lambda h, r: (h * _GRID_R + r, 0)),
        out_shape=jax.ShapeDtypeStruct((NSC * NP, DH), jnp.float32),
    )(xp, W1, b1.reshape(1, F_IN))


def _ew_stage_body(cs0, ck, cu, ax_ref, u_ref, y_ref, s_ref, h_ref, al_ref,
                   s_out, u_out):
    sig = 0.5 * jax.nn.sigmoid(al_ref[...])
    k = sig * (ax_ref[...] - u_ref[...]) + h_ref[...]
    s_out[...] = cs0 * s_ref[...] + ck * k
    u_out[...] = y_ref[...] + cu * k


def _ew_stage(cs0, ck, cu, ax, u, y, s, h, alpha):
    blk = pl.BlockSpec((_ROW_BLK, DH), lambda b: (b, 0))
    al_blk = pl.BlockSpec((_ROW_BLK, 1), lambda b: (b % _GRID_R, 0))
    return pl.pallas_call(
        functools.partial(_ew_stage_body, cs0, ck, cu),
        grid=(NSC * _GRID_R,),
        in_specs=[blk, blk, blk, blk, blk, al_blk],
        out_specs=[blk, blk],
        out_shape=[
            jax.ShapeDtypeStruct((NSC * NP, DH), jnp.float32),
            jax.ShapeDtypeStruct((NSC * NP, DH), jnp.float32),
        ],
    )(ax, u, y, s, h, alpha)


def _ew_final_body(ax_ref, u_ref, y_ref, s_ref, h_ref, al_ref, y_out):
    sig = 0.5 * jax.nn.sigmoid(al_ref[...])
    k = sig * (ax_ref[...] - u_ref[...]) + h_ref[...]
    y_out[...] = y_ref[...] + (DT / 6.0) * (s_ref[...] + k)


def _ew_final(ax, u, y, s, h, alpha):
    blk = pl.BlockSpec((_ROW_BLK, DH), lambda b: (b, 0))
    al_blk = pl.BlockSpec((_ROW_BLK, 1), lambda b: (b % _GRID_R, 0))
    return pl.pallas_call(
        _ew_final_body,
        grid=(NSC * _GRID_R,),
        in_specs=[blk, blk, blk, blk, blk, al_blk],
        out_specs=blk,
        out_shape=jax.ShapeDtypeStruct((NSC * NP, DH), jnp.float32),
    )(ax, u, y, s, h, alpha)


def _m2_body(ya_ref, yb_ref, w2_ref, b2_ref, out_ref):
    z = jax.nn.relu(jnp.concatenate([ya_ref[...], yb_ref[...]], axis=1))
    out_ref[...] = (
        jnp.dot(z, w2_ref[...], preferred_element_type=jnp.float32) + b2_ref[...]
    )


def _m2(y2, W2, b2):
    return pl.pallas_call(
        _m2_body,
        grid=(_GRID_R,),
        in_specs=[
            pl.BlockSpec((_ROW_BLK, DH), lambda r: (r, 0)),
            pl.BlockSpec((_ROW_BLK, DH), lambda r: (r + _GRID_R, 0)),
            pl.BlockSpec((H, C_OUT), lambda r: (0, 0)),
            pl.BlockSpec((1, C_OUT), lambda r: (0, 0)),
        ],
        out_specs=pl.BlockSpec((_ROW_BLK, C_OUT), lambda r: (r, 0)),
        out_shape=jax.ShapeDtypeStruct((N, C_OUT), jnp.float32),
    )(y2, y2, W2, b2.reshape(1, C_OUT))


def kernel(x, edge_index, edge_weight, W1, b1, alpha_train, W2, b2):
    row = edge_index[0]
    col = edge_index[1]
    pad = EP - E
    colp = jnp.concatenate([col, jnp.zeros((pad,), jnp.int32)])
    rowp = jnp.concatenate([row, jnp.zeros((pad,), jnp.int32)])
    wp = jnp.concatenate([edge_weight, jnp.zeros((pad,), jnp.float32)])
    colp2 = jnp.concatenate([colp, colp + NP])
    rowp4 = rowp.reshape(NT * 2, NCH2, CH)
    xp = jnp.zeros((NP, F_IN), jnp.float32).at[:N].set(x)
    alpha = jnp.zeros((NP, 1), jnp.float32).at[:N, 0].set(alpha_train)

    h2 = _m1(xp, W1, b1)

    def _step(_, y):
        ax = _spmm(y, colp2, rowp4, wp)
        s, u = _ew_stage(0.0, 1.0, DT / 2.0, ax, y, y, y, h2, alpha)
        ax = _spmm(u, colp2, rowp4, wp)
        s, u = _ew_stage(1.0, 2.0, DT / 2.0, ax, u, y, s, h2, alpha)
        ax = _spmm(u, colp2, rowp4, wp)
        s, u = _ew_stage(1.0, 2.0, DT, ax, u, y, s, h2, alpha)
        ax = _spmm(u, colp2, rowp4, wp)
        return _ew_final(ax, u, y, s, h2, alpha)

    y = lax.fori_loop(0, STEPS, _step, h2)
    return _m2(y, W2, b2)


# speedup vs baseline: 1.6527x; 1.1833x over previous
import functools

import jax
import jax.numpy as jnp
from jax import lax
from jax.experimental import pallas as pl
from jax.experimental.pallas import tpu as pltpu
from jax.experimental.pallas import tpu_sc as plsc

N = 10000
NP = 10240
E = 160000
F_IN = 256
H = 256
C_OUT = 40
DH = 128
NSC = 2
NT = 16
CH = 128
ET = 10240
EP = NT * ET
NCH = ET // CH
RPT = NP // NT
T_TOTAL = 1.0
STEPS = 8
DT = T_TOTAL / STEPS

_ROW_BLK = 1024
_GRID_R = NP // _ROW_BLK


def _scale_chunk(gb, wall, j):
    def _grp(g, carry):
        w16 = wall[pl.ds(j * CH + 16 * g, 16)]
        for k in range(16):
            e = g * 16 + k
            wv = w16[k]
            for f in range(8):
                gb[e, pl.ds(16 * f, 16)] = gb[e, pl.ds(16 * f, 16)] * wv
        return carry

    lax.fori_loop(0, CH // 16, _grp, 0)


ET2 = ET // 2
NCH2 = NCH // 2


def _spmm_body(y2, colp2, rowp4, wp, out2, acc, g0, g1, call, rall, wall,
               sg0, sg1):
    c = lax.axis_index("c")
    s = lax.axis_index("s")
    coff = c * NP

    zeros16 = jnp.zeros((16,), jnp.float32)

    def _zrow(i, carry):
        for f in range(8):
            g0[i, pl.ds(16 * f, 16)] = zeros16
        return carry

    lax.fori_loop(0, CH, _zrow, 0)
    for j in range(RPT // CH):
        pltpu.sync_copy(g0, acc.at[pl.ds(s * RPT + j * CH, CH)])
    plsc.subcore_barrier()

    gbufs = (g0, g1)
    gsems = (sg0, sg1)

    for p in range(2):
        pltpu.sync_copy(colp2.at[pl.ds(c * EP + s * ET + p * ET2, ET2)], call)
        pltpu.sync_copy(rowp4.at[s * 2 + p], rall)
        pltpu.sync_copy(wp.at[pl.ds(s * ET + p * ET2, ET2)], wall)
        pltpu.async_copy(y2.at[call.at[pl.ds(0 * CH, CH)]], g0, sg0)
        pltpu.async_copy(y2.at[call.at[pl.ds(1 * CH, CH)]], g1, sg1)

        def _pair(i2, carry):
            for b in range(2):
                j = i2 * 2 + b
                gb, sg = gbufs[b], gsems[b]
                pltpu.make_async_copy(y2.at[pl.ds(0, CH)], gb, sg).wait()
                _scale_chunk(gb, wall, j)
                pltpu.sync_copy(gb, acc.at[rall.at[j]], add=True)

                @pl.when(j + 2 < NCH2)
                def _prefetch():
                    pltpu.async_copy(
                        y2.at[call.at[pl.ds((j + 2) * CH, CH)]], gb, sg)

            return carry

        lax.fori_loop(0, NCH2 // 2, _pair, 0)

    plsc.subcore_barrier()

    for j in range(RPT // CH):
        r0 = s * RPT + j * CH
        pltpu.sync_copy(acc.at[pl.ds(r0, CH)], g0)
        pltpu.sync_copy(g0, out2.at[pl.ds(coff + r0, CH)])


_spmm_cached = None


def _spmm(y2, colp2, rowp4, wp):
    global _spmm_cached
    if _spmm_cached is None:
        _spmm_cached = pl.kernel(
            _spmm_body,
            out_type=jax.ShapeDtypeStruct((NSC * NP, DH), jnp.float32),
            mesh=plsc.VectorSubcoreMesh(core_axis_name="c", subcore_axis_name="s",
                                        num_cores=NSC, num_subcores=NT),
            scratch_types=[
                pltpu.VMEM_SHARED((NP, DH), jnp.float32),
                pltpu.VMEM((CH, DH), jnp.float32),
                pltpu.VMEM((CH, DH), jnp.float32),
                pltpu.VMEM((ET2,), jnp.int32),
                pltpu.VMEM((NCH2, CH), jnp.int32),
                pltpu.VMEM((ET2,), jnp.float32),
                pltpu.SemaphoreType.DMA,
                pltpu.SemaphoreType.DMA,
            ],
        )
    return _spmm_cached(y2, colp2, rowp4, wp)


def _m1_body(x_ref, w1_ref, b1_ref, out_ref):
    out_ref[...] = (
        jnp.dot(x_ref[...], w1_ref[...], preferred_element_type=jnp.float32)
        + b1_ref[...]
    )


def _m1(xp, W1, b1):
    return pl.pallas_call(
        _m1_body,
        grid=(NSC, _GRID_R),
        in_specs=[
            pl.BlockSpec((_ROW_BLK, F_IN), lambda h, r: (r, 0)),
            pl.BlockSpec((F_IN, DH), lambda h, r: (0, h)),
            pl.BlockSpec((1, DH), lambda h, r: (0, h)),
        ],
        out_specs=pl.BlockSpec((_ROW_BLK, DH), lambda h, r: (h * _GRID_R + r, 0)),
        out_shape=jax.ShapeDtypeStruct((NSC * NP, DH), jnp.float32),
    )(xp, W1, b1.reshape(1, F_IN))


def _ew_stage_body(cs0, ck, cu, ax_ref, u_ref, y_ref, s_ref, h_ref, al_ref,
                   s_out, u_out):
    sig = 0.5 * jax.nn.sigmoid(al_ref[...])
    k = sig * (ax_ref[...] - u_ref[...]) + h_ref[...]
    s_out[...] = cs0 * s_ref[...] + ck * k
    u_out[...] = y_ref[...] + cu * k


def _ew_stage(cs0, ck, cu, ax, u, y, s, h, alpha):
    blk = pl.BlockSpec((_ROW_BLK, DH), lambda b: (b, 0))
    al_blk = pl.BlockSpec((_ROW_BLK, 1), lambda b: (b % _GRID_R, 0))
    return pl.pallas_call(
        functools.partial(_ew_stage_body, cs0, ck, cu),
        grid=(NSC * _GRID_R,),
        in_specs=[blk, blk, blk, blk, blk, al_blk],
        out_specs=[blk, blk],
        out_shape=[
            jax.ShapeDtypeStruct((NSC * NP, DH), jnp.float32),
            jax.ShapeDtypeStruct((NSC * NP, DH), jnp.float32),
        ],
    )(ax, u, y, s, h, alpha)


def _ew_final_body(ax_ref, u_ref, y_ref, s_ref, h_ref, al_ref, y_out):
    sig = 0.5 * jax.nn.sigmoid(al_ref[...])
    k = sig * (ax_ref[...] - u_ref[...]) + h_ref[...]
    y_out[...] = y_ref[...] + (DT / 6.0) * (s_ref[...] + k)


def _ew_final(ax, u, y, s, h, alpha):
    blk = pl.BlockSpec((_ROW_BLK, DH), lambda b: (b, 0))
    al_blk = pl.BlockSpec((_ROW_BLK, 1), lambda b: (b % _GRID_R, 0))
    return pl.pallas_call(
        _ew_final_body,
        grid=(NSC * _GRID_R,),
        in_specs=[blk, blk, blk, blk, blk, al_blk],
        out_specs=blk,
        out_shape=jax.ShapeDtypeStruct((NSC * NP, DH), jnp.float32),
    )(ax, u, y, s, h, alpha)


def _m2_body(ya_ref, yb_ref, w2_ref, b2_ref, out_ref):
    z = jax.nn.relu(jnp.concatenate([ya_ref[...], yb_ref[...]], axis=1))
    out_ref[...] = (
        jnp.dot(z, w2_ref[...], preferred_element_type=jnp.float32) + b2_ref[...]
    )


def _m2(y2, W2, b2):
    return pl.pallas_call(
        _m2_body,
        grid=(_GRID_R,),
        in_specs=[
            pl.BlockSpec((_ROW_BLK, DH), lambda r: (r, 0)),
            pl.BlockSpec((_ROW_BLK, DH), lambda r: (r + _GRID_R, 0)),
            pl.BlockSpec((H, C_OUT), lambda r: (0, 0)),
            pl.BlockSpec((1, C_OUT), lambda r: (0, 0)),
        ],
        out_specs=pl.BlockSpec((_ROW_BLK, C_OUT), lambda r: (r, 0)),
        out_shape=jax.ShapeDtypeStruct((N, C_OUT), jnp.float32),
    )(y2, y2, W2, b2.reshape(1, C_OUT))


def kernel(x, edge_index, edge_weight, W1, b1, alpha_train, W2, b2):
    row = edge_index[0]
    col = edge_index[1]
    pad = EP - E
    colp = jnp.concatenate([col, jnp.zeros((pad,), jnp.int32)])
    rowp = jnp.concatenate([row, jnp.zeros((pad,), jnp.int32)])
    wp = jnp.concatenate([edge_weight, jnp.zeros((pad,), jnp.float32)])
    colp2 = jnp.concatenate([colp, colp + NP])
    rowp4 = rowp.reshape(NT * 2, NCH2, CH)
    xp = jnp.zeros((NP, F_IN), jnp.float32).at[:N].set(x)
    alpha = jnp.zeros((NP, 1), jnp.float32).at[:N, 0].set(alpha_train)

    h2 = _m1(xp, W1, b1)

    def _step(_, y):
        ax = _spmm(y, colp2, rowp4, wp)
        s, u = _ew_stage(0.0, 1.0, DT / 2.0, ax, y, y, y, h2, alpha)
        ax = _spmm(u, colp2, rowp4, wp)
        s, u = _ew_stage(1.0, 2.0, DT / 2.0, ax, u, y, s, h2, alpha)
        ax = _spmm(u, colp2, rowp4, wp)
        s, u = _ew_stage(1.0, 2.0, DT, ax, u, y, s, h2, alpha)
        ax = _spmm(u, colp2, rowp4, wp)
        return _ew_final(ax, u, y, s, h2, alpha)

    y = lax.fori_loop(0, STEPS, _step, h2)
    return _m2(y, W2, b2)
